# Initial kernel scaffold; baseline (speedup 1.0000x reference)
import jax
import jax.numpy as jnp
from jax.experimental import pallas as pl


def kernel(seqs, attention_mask, time_matrices, W1, b1, W2, b2, Ww, Wb, ln_g, ln_b):
    raise NotImplementedError("write your pallas kernel here")


# speedup vs baseline: 1.4636x; 1.4636x over previous
import jax
import jax.numpy as jnp
from jax.experimental import pallas as pl
from jax.experimental.pallas import tpu as pltpu

_B, _L, _D, _H = 128, 50, 128, 2
_HS = _D // _H


def _bf(x):
    return x.astype(jnp.bfloat16).astype(jnp.float32)


def _core(seqs_ref, amask_ref, tm_ref, W1_ref, b1_ref, W2_ref, b2_ref,
          Ww_ref, Wb_ref, g_ref, beta_ref, out_ref, tio_ref):
    s16 = seqs_ref[0].astype(jnp.bfloat16)
    amask = amask_ref[...] != 0
    tm = tm_ref[0]

    a_full = jax.lax.dot_general(s16, W1_ref[...].astype(jnp.bfloat16),
                                 (((1,), (1,)), ((), ())),
                                 preferred_element_type=jnp.float32) + b1_ref[...]
    b_full = jax.lax.dot_general(s16, W2_ref[...].astype(jnp.bfloat16),
                                 (((1,), (1,)), ((), ())),
                                 preferred_element_type=jnp.float32) + b2_ref[...]
    v_full = jax.lax.dot_general(s16, Ww_ref[...].astype(jnp.bfloat16),
                                 (((1,), (1,)), ((), ())),
                                 preferred_element_type=jnp.float32) + Wb_ref[...]

    iota = jax.lax.broadcasted_iota(jnp.int32, (_L, _L), 1)
    outs = []
    tios = []
    for h in range(_H):
        sl = slice(h * _HS, (h + 1) * _HS)
        a = a_full[:, sl]
        bh = b_full[:, sl]
        vv = v_full[:, sl]
        ti = tm[:, :, sl]
        a16 = _bf(a)
        ti16 = _bf(ti)

        att0 = jax.lax.dot_general(a.astype(jnp.bfloat16), bh.astype(jnp.bfloat16),
                                   (((1,), (1,)), ((), ())),
                                   preferred_element_type=jnp.float32)
        ti_a = jnp.sum(ti16 * a16[:, None, :], axis=-1)
        att = att0 + ti_a

        bt = ti + bh[None, :, :]
        bt2 = jnp.sqrt(jnp.sum(bt * bt, axis=-1))
        a2 = jnp.sqrt(jnp.sum(a * a, axis=-1))
        raw = att / (a2[:, None] * bt2 + 1e-6)
        raw = jnp.where(amask, 0.0, raw)

        r = raw
        M = jnp.zeros((_L, _L), jnp.float32)
        for _ in range(3):
            mx = jnp.max(r, axis=1, keepdims=True)
            sel = r == mx
            jmin = jnp.min(jnp.where(sel, iota, _L), axis=1, keepdims=True)
            onehot = iota == jmin
            M = jnp.maximum(M, onehot.astype(jnp.float32))
            r = jnp.where(onehot, -jnp.inf, r)
        mask = jnp.maximum(M, M.T)
        sparse = raw * mask
        sparse = jnp.where(amask, 0.0, sparse)

        out_h = jax.lax.dot(sparse.astype(jnp.bfloat16), vv.astype(jnp.bfloat16),
                            preferred_element_type=jnp.float32)
        tio_h = jnp.sum(_bf(sparse)[:, :, None] * ti16, axis=1)
        outs.append(out_h)
        tios.append(tio_h)

    out = jnp.concatenate(outs, axis=-1)
    mu = jnp.mean(out, axis=-1, keepdims=True)
    var = jnp.mean((out - mu) ** 2, axis=-1, keepdims=True)
    out_ref[0] = (out - mu) / jnp.sqrt(var + 1e-8) * g_ref[...] + beta_ref[...]
    tio_ref[0] = jnp.concatenate(tios, axis=-1)


def kernel(seqs, attention_mask, time_matrices, W1, b1, W2, b2, Ww, Wb, ln_g, ln_b):
    amask_f = attention_mask.astype(jnp.float32)
    b1r = b1.reshape(1, _D)
    b2r = b2.reshape(1, _D)
    Wbr = Wb.reshape(1, _D)
    gr = ln_g.reshape(1, _D)
    br = ln_b.reshape(1, _D)

    const2d = pl.BlockSpec((_L, _L), lambda b: (0, 0))
    w_spec = pl.BlockSpec((_D, _D), lambda b: (0, 0))
    v_spec = pl.BlockSpec((1, _D), lambda b: (0, 0))

    out, tio = pl.pallas_call(
        _core,
        grid=(_B,),
        in_specs=[
            pl.BlockSpec((1, _L, _D), lambda b: (b, 0, 0)),
            const2d,
            pl.BlockSpec((1, _L, _L, _D), lambda b: (b, 0, 0, 0)),
            w_spec, v_spec, w_spec, v_spec, w_spec, v_spec, v_spec, v_spec,
        ],
        out_specs=[
            pl.BlockSpec((1, _L, _D), lambda b: (b, 0, 0)),
            pl.BlockSpec((1, _L, _D), lambda b: (b, 0, 0)),
        ],
        out_shape=[
            jax.ShapeDtypeStruct((_B, _L, _D), jnp.float32),
            jax.ShapeDtypeStruct((_B, _L, _D), jnp.float32),
        ],
    )(seqs, amask_f, time_matrices, W1, b1r, W2, b2r, Ww, Wbr, gr, br)
    return out, tio
